# Initial kernel scaffold; baseline (speedup 1.0000x reference)
import functools

import jax
import jax.numpy as jnp
from jax import lax
from jax.experimental import pallas as pl
from jax.experimental.pallas import tpu as pltpu
from jax.experimental.pallas import tpu_sc as plsc

N = 1_600_000
NUM_SEG = 50_000
EPS = 0.001

NC = 2
NS = 16
NW = NC * NS
L = 16

SEG_PAD = 51_200
SEG_SLICE = SEG_PAD // NS
N_PAD = 1_638_400
TILE = 2048
ROWS = TILE // 128
K_STEPS = N_PAD // (NW * TILE)
SUB = 800
F32 = jnp.float32
I32 = jnp.int32

_mesh = plsc.VectorSubcoreMesh(core_axis_name="c", subcore_axis_name="s")


def _fill(ref, n, val, dtype):
    v = jnp.full((L,), val, dtype)

    def body(i, _):
        ref[pl.ds(i * L, L)] = v
        return 0

    lax.fori_loop(0, n // L, body, 0)


@functools.partial(
    pl.kernel,
    out_type=jax.ShapeDtypeStruct((NC, 3, SEG_PAD), F32),
    mesh=_mesh,
    scratch_types=[
        pltpu.VMEM((TILE,), F32),
        pltpu.VMEM((TILE,), F32),
        pltpu.VMEM((ROWS, 128), I32),
        pltpu.VMEM((128,), F32),
        pltpu.VMEM((SEG_SLICE,), F32),
        pltpu.VMEM_SHARED((SEG_PAD,), F32),
        pltpu.VMEM_SHARED((SEG_PAD,), F32),
        pltpu.VMEM_SHARED((SEG_PAD,), F32),
    ],
)
def _stats(x_hbm, b2d_hbm, part_hbm, xbuf, xsq, idxbuf, ones, zbuf,
           acc_c, acc_s, acc_q):
    c = lax.axis_index("c")
    s = lax.axis_index("s")
    w = c * NS + s
    base = s * SEG_SLICE

    _fill(ones, 128, 1.0, F32)
    _fill(zbuf, SEG_SLICE, 0.0, F32)
    pltpu.sync_copy(zbuf, acc_c.at[pl.ds(base, SEG_SLICE)])
    pltpu.sync_copy(zbuf, acc_s.at[pl.ds(base, SEG_SLICE)])
    pltpu.sync_copy(zbuf, acc_q.at[pl.ds(base, SEG_SLICE)])
    plsc.subcore_barrier()

    def kstep(k, _):
        e0 = (w * K_STEPS + k) * TILE
        r0 = (w * K_STEPS + k) * ROWS
        pltpu.sync_copy(x_hbm.at[pl.ds(e0, TILE)], xbuf)
        pltpu.sync_copy(b2d_hbm.at[pl.ds(r0, ROWS)], idxbuf)

        def sq(i, _):
            v = xbuf[pl.ds(i * L, L)]
            xsq[pl.ds(i * L, L)] = v * v
            return 0

        lax.fori_loop(0, TILE // L, sq, 0)

        def row(j, _):
            idxr = idxbuf.at[j]
            pltpu.sync_copy(ones, acc_c.at[idxr], add=True)
            pltpu.sync_copy(xbuf.at[pl.ds(j * 128, 128)], acc_s.at[idxr], add=True)
            pltpu.sync_copy(xsq.at[pl.ds(j * 128, 128)], acc_q.at[idxr], add=True)
            return 0

        lax.fori_loop(0, ROWS, row, 0)
        return 0

    lax.fori_loop(0, K_STEPS, kstep, 0)
    plsc.subcore_barrier()

    pltpu.sync_copy(acc_c.at[pl.ds(base, SEG_SLICE)], zbuf)
    pltpu.sync_copy(zbuf, part_hbm.at[c, 0, pl.ds(base, SEG_SLICE)])
    pltpu.sync_copy(acc_s.at[pl.ds(base, SEG_SLICE)], zbuf)
    pltpu.sync_copy(zbuf, part_hbm.at[c, 1, pl.ds(base, SEG_SLICE)])
    pltpu.sync_copy(acc_q.at[pl.ds(base, SEG_SLICE)], zbuf)
    pltpu.sync_copy(zbuf, part_hbm.at[c, 2, pl.ds(base, SEG_SLICE)])


def _rsqrt(v):
    bits = plsc.bitcast(v, I32)
    magic = jnp.full((L,), 0x5F3759DF, I32)
    shift = jnp.full((L,), 1, I32)
    y = plsc.bitcast(magic - lax.shift_right_logical(bits, shift), F32)
    half = jnp.float32(0.5)
    three_half = jnp.float32(1.5)
    for _ in range(3):
        y = y * (three_half - half * v * y * y)
    return y


@functools.partial(
    pl.kernel,
    out_type=jax.ShapeDtypeStruct((N_PAD,), F32),
    mesh=_mesh,
    scratch_types=[
        pltpu.VMEM((SEG_PAD,), F32),
        pltpu.VMEM((SEG_PAD,), F32),
        pltpu.VMEM((SUB,), F32),
        pltpu.VMEM((SUB,), F32),
        pltpu.VMEM((SUB,), F32),
        pltpu.VMEM((SUB,), F32),
        pltpu.VMEM((SUB,), F32),
        pltpu.VMEM((SUB,), F32),
        pltpu.VMEM((TILE,), F32),
        pltpu.VMEM((TILE,), I32),
        pltpu.VMEM((TILE,), F32),
        pltpu.VMEM((L,), F32),
        pltpu.VMEM((L,), F32),
        pltpu.VMEM_SHARED((SEG_PAD,), F32),
        pltpu.VMEM_SHARED((SEG_PAD,), F32),
    ],
)
def _norm(x_hbm, b_hbm, part_hbm, gain_hbm, bias_hbm, out_hbm,
          mean_v, scale_v, p0, p1, p2, p3, p4, p5, xbuf, idxbuf, obuf,
          gb, bb, mean_sp, scale_sp):
    c = lax.axis_index("c")
    s = lax.axis_index("s")
    w = c * NS + s

    pltpu.sync_copy(gain_hbm, gb)
    pltpu.sync_copy(bias_hbm, bb)
    gain = gb[...]
    bias = bb[...]

    for sub in range(SEG_SLICE // SUB):
        sb = s * SEG_SLICE + sub * SUB
        pltpu.sync_copy(part_hbm.at[0, 0, pl.ds(sb, SUB)], p0)
        pltpu.sync_copy(part_hbm.at[1, 0, pl.ds(sb, SUB)], p1)
        pltpu.sync_copy(part_hbm.at[0, 1, pl.ds(sb, SUB)], p2)
        pltpu.sync_copy(part_hbm.at[1, 1, pl.ds(sb, SUB)], p3)
        pltpu.sync_copy(part_hbm.at[0, 2, pl.ds(sb, SUB)], p4)
        pltpu.sync_copy(part_hbm.at[1, 2, pl.ds(sb, SUB)], p5)

        def seg(i, _):
            d = pl.ds(i * L, L)
            cnt = p0[d] + p1[d]
            cl = jnp.maximum(cnt, jnp.float32(1.0))
            sm = p2[d] + p3[d]
            q = p4[d] + p5[d]
            m = sm / cl
            var = jnp.maximum(q / cl - m * m, jnp.float32(0.0))
            vs = jnp.maximum(var, jnp.float32(1e-30))
            std = vs * _rsqrt(vs)
            g = pl.ds(sb + i * L, L)
            mean_v[g] = m
            scale_v[g] = gain / (std + jnp.float32(EPS))
            return 0

        lax.fori_loop(0, SUB // L, seg, 0)

    sl = pl.ds(s * SEG_SLICE, SEG_SLICE)
    pltpu.sync_copy(mean_v.at[sl], mean_sp.at[sl])
    pltpu.sync_copy(scale_v.at[sl], scale_sp.at[sl])
    plsc.subcore_barrier()
    pltpu.sync_copy(mean_sp, mean_v)
    pltpu.sync_copy(scale_sp, scale_v)

    def kstep(k, _):
        e0 = (w * K_STEPS + k) * TILE
        pltpu.sync_copy(x_hbm.at[pl.ds(e0, TILE)], xbuf)
        pltpu.sync_copy(b_hbm.at[pl.ds(e0, TILE)], idxbuf)

        def inner(i, _):
            d = pl.ds(i * L, L)
            idxv = idxbuf[d]
            xv = xbuf[d]
            m = plsc.load_gather(mean_v, [idxv])
            g = plsc.load_gather(scale_v, [idxv])
            obuf[d] = (xv - m) * g + bias
            return 0

        lax.fori_loop(0, TILE // L, inner, 0)
        pltpu.sync_copy(obuf, out_hbm.at[pl.ds(e0, TILE)])
        return 0

    lax.fori_loop(0, K_STEPS, kstep, 0)


def kernel(inputs, batch, gain, bias):
    x = inputs.astype(F32)
    b = batch.astype(I32)
    pad = N_PAD - N
    xp = jnp.concatenate([x, jnp.zeros((pad,), F32)])
    dummy = NUM_SEG + (jnp.arange(pad, dtype=I32) % (SEG_PAD - NUM_SEG))
    bp = jnp.concatenate([b, dummy])
    b2d = bp.reshape(N_PAD // 128, 128)
    g16 = jnp.broadcast_to(gain.astype(F32), (L,))
    b16 = jnp.broadcast_to(bias.astype(F32), (L,))
    part = _stats(xp, b2d)
    outp = _norm(xp, bp, part, g16, b16)
    return outp[:N].reshape(N, 1)


# speedup vs baseline: 87.4475x; 87.4475x over previous
import functools

import jax
import jax.numpy as jnp
from jax import lax
from jax.experimental import pallas as pl
from jax.experimental.pallas import tpu as pltpu
from jax.experimental.pallas import tpu_sc as plsc

N = 1_600_000
NUM_SEG = 50_000
EPS = 0.001

NC = 2
NS = 16
NW = NC * NS
L = 16

SEG_PAD = 51_200
SEG_SLICE = SEG_PAD // NS
N_PAD = 1_638_400
TILE = 2048
ROWS = TILE // 128
K_STEPS = N_PAD // (NW * TILE)
SUB = 800
F32 = jnp.float32
I32 = jnp.int32

_mesh = plsc.VectorSubcoreMesh(core_axis_name="c", subcore_axis_name="s")


def _fill(ref, n, val, dtype):
    v = jnp.full((L,), val, dtype)

    def body(i, _):
        ref[pl.ds(i * L, L)] = v
        return 0

    lax.fori_loop(0, n // L, body, 0)


@functools.partial(
    pl.kernel,
    out_type=jax.ShapeDtypeStruct((NC * 3 * SEG_PAD,), F32),
    mesh=_mesh,
    scratch_types=[
        pltpu.VMEM((TILE,), F32),
        pltpu.VMEM((TILE,), F32),
        pltpu.VMEM((ROWS, 128), I32),
        pltpu.VMEM((128,), F32),
        pltpu.VMEM((SEG_SLICE,), F32),
        pltpu.VMEM_SHARED((SEG_PAD,), F32),
        pltpu.VMEM_SHARED((SEG_PAD,), F32),
        pltpu.VMEM_SHARED((SEG_PAD,), F32),
    ],
)
def _stats(x_hbm, b2d_hbm, part_hbm, xbuf, xsq, idxbuf, ones, zbuf,
           acc_c, acc_s, acc_q):
    c = lax.axis_index("c")
    s = lax.axis_index("s")
    w = c * NS + s
    base = s * SEG_SLICE

    _fill(ones, 128, 1.0, F32)
    _fill(zbuf, SEG_SLICE, 0.0, F32)
    pltpu.sync_copy(zbuf, acc_c.at[pl.ds(base, SEG_SLICE)])
    pltpu.sync_copy(zbuf, acc_s.at[pl.ds(base, SEG_SLICE)])
    pltpu.sync_copy(zbuf, acc_q.at[pl.ds(base, SEG_SLICE)])
    plsc.subcore_barrier()

    def kstep(k, _):
        e0 = (w * K_STEPS + k) * TILE
        r0 = (w * K_STEPS + k) * ROWS
        pltpu.sync_copy(x_hbm.at[pl.ds(e0, TILE)], xbuf)
        pltpu.sync_copy(b2d_hbm.at[pl.ds(r0, ROWS)], idxbuf)

        def sq(i, _):
            v = xbuf[pl.ds(i * L, L)]
            xsq[pl.ds(i * L, L)] = v * v
            return 0

        lax.fori_loop(0, TILE // L, sq, 0)

        def row(j, _):
            idxr = idxbuf.at[j]
            pltpu.sync_copy(ones, acc_c.at[idxr], add=True)
            pltpu.sync_copy(xbuf.at[pl.ds(j * 128, 128)], acc_s.at[idxr], add=True)
            pltpu.sync_copy(xsq.at[pl.ds(j * 128, 128)], acc_q.at[idxr], add=True)
            return 0

        lax.fori_loop(0, ROWS, row, 0)
        return 0

    lax.fori_loop(0, K_STEPS, kstep, 0)
    plsc.subcore_barrier()

    pltpu.sync_copy(acc_c.at[pl.ds(base, SEG_SLICE)], zbuf)
    pltpu.sync_copy(zbuf, part_hbm.at[pl.ds(c * 3 * SEG_PAD + base, SEG_SLICE)])
    pltpu.sync_copy(acc_s.at[pl.ds(base, SEG_SLICE)], zbuf)
    pltpu.sync_copy(zbuf, part_hbm.at[pl.ds((c * 3 + 1) * SEG_PAD + base, SEG_SLICE)])
    pltpu.sync_copy(acc_q.at[pl.ds(base, SEG_SLICE)], zbuf)
    pltpu.sync_copy(zbuf, part_hbm.at[pl.ds((c * 3 + 2) * SEG_PAD + base, SEG_SLICE)])


def _rsqrt(v):
    bits = lax.bitcast_convert_type(v, I32)
    magic = jnp.full((L,), 0x5F3759DF, I32)
    shift = jnp.full((L,), 1, I32)
    y = lax.bitcast_convert_type(magic - lax.shift_right_logical(bits, shift), F32)
    half = jnp.float32(0.5)
    three_half = jnp.float32(1.5)
    for _ in range(3):
        y = y * (three_half - half * v * y * y)
    return y


@functools.partial(
    pl.kernel,
    out_type=jax.ShapeDtypeStruct((N_PAD,), F32),
    mesh=_mesh,
    scratch_types=[
        pltpu.VMEM((SEG_SLICE,), F32),
        pltpu.VMEM((SEG_SLICE,), F32),
        pltpu.VMEM((SUB,), F32),
        pltpu.VMEM((SUB,), F32),
        pltpu.VMEM((SUB,), F32),
        pltpu.VMEM((SUB,), F32),
        pltpu.VMEM((SUB,), F32),
        pltpu.VMEM((SUB,), F32),
        pltpu.VMEM((TILE,), F32),
        pltpu.VMEM((TILE,), I32),
        pltpu.VMEM((TILE,), F32),
        pltpu.VMEM((TILE,), F32),
        pltpu.VMEM((TILE,), F32),
        pltpu.VMEM((L,), F32),
        pltpu.VMEM((L,), F32),
        pltpu.VMEM_SHARED((SEG_PAD,), F32),
        pltpu.VMEM_SHARED((SEG_PAD,), F32),
    ],
)
def _norm(x_hbm, b_hbm, part_hbm, gain_hbm, bias_hbm, out_hbm,
          a_v, b_v, p0, p1, p2, p3, p4, p5, xbuf, idxbuf, abuf, bbuf, obuf,
          gb, bb, a_sp, b_sp):
    c = lax.axis_index("c")
    s = lax.axis_index("s")
    w = c * NS + s

    pltpu.sync_copy(gain_hbm, gb)
    pltpu.sync_copy(bias_hbm, bb)
    gain = gb[...]
    bias = bb[...]

    for sub in range(SEG_SLICE // SUB):
        sb = s * SEG_SLICE + sub * SUB
        pltpu.sync_copy(part_hbm.at[pl.ds(0 * SEG_PAD + sb, SUB)], p0)
        pltpu.sync_copy(part_hbm.at[pl.ds(3 * SEG_PAD + sb, SUB)], p1)
        pltpu.sync_copy(part_hbm.at[pl.ds(1 * SEG_PAD + sb, SUB)], p2)
        pltpu.sync_copy(part_hbm.at[pl.ds(4 * SEG_PAD + sb, SUB)], p3)
        pltpu.sync_copy(part_hbm.at[pl.ds(2 * SEG_PAD + sb, SUB)], p4)
        pltpu.sync_copy(part_hbm.at[pl.ds(5 * SEG_PAD + sb, SUB)], p5)

        def seg(i, _):
            d = pl.ds(i * L, L)
            cnt = p0[d] + p1[d]
            cl = jnp.maximum(cnt, jnp.float32(1.0))
            sm = p2[d] + p3[d]
            q = p4[d] + p5[d]
            m = sm / cl
            var = jnp.maximum(q / cl - m * m, jnp.float32(0.0))
            vs = jnp.maximum(var, jnp.float32(1e-30))
            std = vs * _rsqrt(vs)
            a = gain / (std + jnp.float32(EPS))
            g = pl.ds(sub * SUB + i * L, L)
            a_v[g] = a
            b_v[g] = bias - m * a
            return 0

        lax.fori_loop(0, SUB // L, seg, 0)

    sl = pl.ds(s * SEG_SLICE, SEG_SLICE)
    pltpu.sync_copy(a_v, a_sp.at[sl])
    pltpu.sync_copy(b_v, b_sp.at[sl])
    plsc.subcore_barrier()

    def kstep(k, _):
        e0 = (w * K_STEPS + k) * TILE
        pltpu.sync_copy(x_hbm.at[pl.ds(e0, TILE)], xbuf)
        pltpu.sync_copy(b_hbm.at[pl.ds(e0, TILE)], idxbuf)

        def row(j, _):
            d = pl.ds(j * 128, 128)
            idxr = idxbuf.at[d]
            pltpu.sync_copy(a_sp.at[idxr], abuf.at[d])
            pltpu.sync_copy(b_sp.at[idxr], bbuf.at[d])
            return 0

        lax.fori_loop(0, ROWS, row, 0)

        def inner(i, _):
            d = pl.ds(i * L, L)
            obuf[d] = xbuf[d] * abuf[d] + bbuf[d]
            return 0

        lax.fori_loop(0, TILE // L, inner, 0)
        pltpu.sync_copy(obuf, out_hbm.at[pl.ds(e0, TILE)])
        return 0

    lax.fori_loop(0, K_STEPS, kstep, 0)


def kernel(inputs, batch, gain, bias):
    x = inputs.astype(F32)
    b = batch.astype(I32)
    pad = N_PAD - N
    xp = jnp.concatenate([x, jnp.zeros((pad,), F32)])
    dummy = NUM_SEG + (jnp.arange(pad, dtype=I32) % (SEG_PAD - NUM_SEG))
    bp = jnp.concatenate([b, dummy])
    b2d = bp.reshape(N_PAD // 128, 128)
    g16 = jnp.broadcast_to(gain.astype(F32), (L,))
    b16 = jnp.broadcast_to(bias.astype(F32), (L,))
    part = _stats(xp, b2d)
    outp = _norm(xp, bp, part, g16, b16)
    return outp[:N].reshape(N, 1)


# speedup vs baseline: 117.2356x; 1.3406x over previous
import functools

import jax
import jax.numpy as jnp
from jax import lax
from jax.experimental import pallas as pl
from jax.experimental.pallas import tpu as pltpu
from jax.experimental.pallas import tpu_sc as plsc

N = 1_600_000
NUM_SEG = 50_000
EPS = 0.001

NC = 2
NS = 16
NW = NC * NS
L = 16

SEG_PAD = 51_200
SEG_SLICE = SEG_PAD // NS
N_PAD = 1_638_400
TILE = 2048
ROWS = TILE // 128
K_STEPS = N_PAD // (NW * TILE)
SUB = 800
F32 = jnp.float32
I32 = jnp.int32

_mesh = plsc.VectorSubcoreMesh(core_axis_name="c", subcore_axis_name="s")


def _fill(ref, n, val, dtype):
    v = jnp.full((L,), val, dtype)

    def body(i, _):
        ref[pl.ds(i * L, L)] = v
        return 0

    lax.fori_loop(0, n // L, body, 0)


@functools.partial(
    pl.kernel,
    out_type=jax.ShapeDtypeStruct((NC * 3 * SEG_PAD,), F32),
    mesh=_mesh,
    scratch_types=[
        pltpu.VMEM((TILE,), F32),
        pltpu.VMEM((TILE,), F32),
        pltpu.VMEM((ROWS, 128), I32),
        pltpu.VMEM((128,), F32),
        pltpu.VMEM((SEG_SLICE,), F32),
        pltpu.VMEM_SHARED((SEG_PAD,), F32),
        pltpu.VMEM_SHARED((SEG_PAD,), F32),
        pltpu.VMEM_SHARED((SEG_PAD,), F32),
        pltpu.SemaphoreType.DMA,
    ],
)
def _stats(x_hbm, b2d_hbm, part_hbm, xbuf, xsq, idxbuf, ones, zbuf,
           acc_c, acc_s, acc_q, sem):
    c = lax.axis_index("c")
    s = lax.axis_index("s")
    w = c * NS + s
    base = s * SEG_SLICE

    _fill(ones, 128, 1.0, F32)
    _fill(zbuf, SEG_SLICE, 0.0, F32)
    pltpu.sync_copy(zbuf, acc_c.at[pl.ds(base, SEG_SLICE)])
    pltpu.sync_copy(zbuf, acc_s.at[pl.ds(base, SEG_SLICE)])
    pltpu.sync_copy(zbuf, acc_q.at[pl.ds(base, SEG_SLICE)])
    plsc.subcore_barrier()

    def kstep(k, _):
        e0 = (w * K_STEPS + k) * TILE
        r0 = (w * K_STEPS + k) * ROWS
        pltpu.sync_copy(x_hbm.at[pl.ds(e0, TILE)], xbuf)
        pltpu.sync_copy(b2d_hbm.at[pl.ds(r0, ROWS)], idxbuf)

        def sq(i, _):
            v = xbuf[pl.ds(i * L, L)]
            xsq[pl.ds(i * L, L)] = v * v
            return 0

        lax.fori_loop(0, TILE // L, sq, 0)

        def fire(j, _):
            idxr = idxbuf.at[j]
            pltpu.async_copy(ones, acc_c.at[idxr], sem, add=True)
            pltpu.async_copy(xbuf.at[pl.ds(j * 128, 128)], acc_s.at[idxr], sem, add=True)
            pltpu.async_copy(xsq.at[pl.ds(j * 128, 128)], acc_q.at[idxr], sem, add=True)
            return 0

        lax.fori_loop(0, ROWS, fire, 0)

        def drain(j, _):
            idxr = idxbuf.at[j]
            pltpu.make_async_copy(ones, acc_c.at[idxr], sem).wait()
            pltpu.make_async_copy(xbuf.at[pl.ds(j * 128, 128)], acc_s.at[idxr], sem).wait()
            pltpu.make_async_copy(xsq.at[pl.ds(j * 128, 128)], acc_q.at[idxr], sem).wait()
            return 0

        lax.fori_loop(0, ROWS, drain, 0)
        return 0

    lax.fori_loop(0, K_STEPS, kstep, 0)
    plsc.subcore_barrier()

    pltpu.sync_copy(acc_c.at[pl.ds(base, SEG_SLICE)], zbuf)
    pltpu.sync_copy(zbuf, part_hbm.at[pl.ds(c * 3 * SEG_PAD + base, SEG_SLICE)])
    pltpu.sync_copy(acc_s.at[pl.ds(base, SEG_SLICE)], zbuf)
    pltpu.sync_copy(zbuf, part_hbm.at[pl.ds((c * 3 + 1) * SEG_PAD + base, SEG_SLICE)])
    pltpu.sync_copy(acc_q.at[pl.ds(base, SEG_SLICE)], zbuf)
    pltpu.sync_copy(zbuf, part_hbm.at[pl.ds((c * 3 + 2) * SEG_PAD + base, SEG_SLICE)])


def _rsqrt(v):
    bits = lax.bitcast_convert_type(v, I32)
    magic = jnp.full((L,), 0x5F3759DF, I32)
    shift = jnp.full((L,), 1, I32)
    y = lax.bitcast_convert_type(magic - lax.shift_right_logical(bits, shift), F32)
    half = jnp.float32(0.5)
    three_half = jnp.float32(1.5)
    for _ in range(3):
        y = y * (three_half - half * v * y * y)
    return y


@functools.partial(
    pl.kernel,
    out_type=jax.ShapeDtypeStruct((N_PAD,), F32),
    mesh=_mesh,
    scratch_types=[
        pltpu.VMEM((SEG_SLICE,), F32),
        pltpu.VMEM((SEG_SLICE,), F32),
        pltpu.VMEM((SUB,), F32),
        pltpu.VMEM((SUB,), F32),
        pltpu.VMEM((SUB,), F32),
        pltpu.VMEM((SUB,), F32),
        pltpu.VMEM((SUB,), F32),
        pltpu.VMEM((SUB,), F32),
        pltpu.VMEM((TILE,), F32),
        pltpu.VMEM((TILE,), I32),
        pltpu.VMEM((TILE,), F32),
        pltpu.VMEM((TILE,), F32),
        pltpu.VMEM((TILE,), F32),
        pltpu.VMEM((L,), F32),
        pltpu.VMEM((L,), F32),
        pltpu.VMEM_SHARED((SEG_PAD,), F32),
        pltpu.VMEM_SHARED((SEG_PAD,), F32),
        pltpu.SemaphoreType.DMA,
    ],
)
def _norm(x_hbm, b_hbm, part_hbm, gain_hbm, bias_hbm, out_hbm,
          a_v, b_v, p0, p1, p2, p3, p4, p5, xbuf, idxbuf, abuf, bbuf, obuf,
          gb, bb, a_sp, b_sp, sem):
    c = lax.axis_index("c")
    s = lax.axis_index("s")
    w = c * NS + s

    pltpu.sync_copy(gain_hbm, gb)
    pltpu.sync_copy(bias_hbm, bb)
    gain = gb[...]
    bias = bb[...]

    for sub in range(SEG_SLICE // SUB):
        sb = s * SEG_SLICE + sub * SUB
        pltpu.sync_copy(part_hbm.at[pl.ds(0 * SEG_PAD + sb, SUB)], p0)
        pltpu.sync_copy(part_hbm.at[pl.ds(3 * SEG_PAD + sb, SUB)], p1)
        pltpu.sync_copy(part_hbm.at[pl.ds(1 * SEG_PAD + sb, SUB)], p2)
        pltpu.sync_copy(part_hbm.at[pl.ds(4 * SEG_PAD + sb, SUB)], p3)
        pltpu.sync_copy(part_hbm.at[pl.ds(2 * SEG_PAD + sb, SUB)], p4)
        pltpu.sync_copy(part_hbm.at[pl.ds(5 * SEG_PAD + sb, SUB)], p5)

        def seg(i, _):
            d = pl.ds(i * L, L)
            cnt = p0[d] + p1[d]
            cl = jnp.maximum(cnt, jnp.float32(1.0))
            sm = p2[d] + p3[d]
            q = p4[d] + p5[d]
            m = sm / cl
            var = jnp.maximum(q / cl - m * m, jnp.float32(0.0))
            vs = jnp.maximum(var, jnp.float32(1e-30))
            std = vs * _rsqrt(vs)
            a = gain / (std + jnp.float32(EPS))
            g = pl.ds(sub * SUB + i * L, L)
            a_v[g] = a
            b_v[g] = bias - m * a
            return 0

        lax.fori_loop(0, SUB // L, seg, 0)

    sl = pl.ds(s * SEG_SLICE, SEG_SLICE)
    pltpu.sync_copy(a_v, a_sp.at[sl])
    pltpu.sync_copy(b_v, b_sp.at[sl])
    plsc.subcore_barrier()

    def kstep(k, _):
        e0 = (w * K_STEPS + k) * TILE
        pltpu.sync_copy(x_hbm.at[pl.ds(e0, TILE)], xbuf)
        pltpu.sync_copy(b_hbm.at[pl.ds(e0, TILE)], idxbuf)

        def fire(j, _):
            d = pl.ds(j * 128, 128)
            idxr = idxbuf.at[d]
            pltpu.async_copy(a_sp.at[idxr], abuf.at[d], sem)
            pltpu.async_copy(b_sp.at[idxr], bbuf.at[d], sem)
            return 0

        lax.fori_loop(0, ROWS, fire, 0)

        def drain(j, _):
            d = pl.ds(j * 128, 128)
            idxr = idxbuf.at[d]
            pltpu.make_async_copy(a_sp.at[idxr], abuf.at[d], sem).wait()
            pltpu.make_async_copy(b_sp.at[idxr], bbuf.at[d], sem).wait()
            return 0

        lax.fori_loop(0, ROWS, drain, 0)

        def inner(i, _):
            d = pl.ds(i * L, L)
            obuf[d] = xbuf[d] * abuf[d] + bbuf[d]
            return 0

        lax.fori_loop(0, TILE // L, inner, 0)
        pltpu.sync_copy(obuf, out_hbm.at[pl.ds(e0, TILE)])
        return 0

    lax.fori_loop(0, K_STEPS, kstep, 0)


def kernel(inputs, batch, gain, bias):
    x = inputs.astype(F32)
    b = batch.astype(I32)
    pad = N_PAD - N
    xp = jnp.concatenate([x, jnp.zeros((pad,), F32)])
    dummy = NUM_SEG + (jnp.arange(pad, dtype=I32) % (SEG_PAD - NUM_SEG))
    bp = jnp.concatenate([b, dummy])
    b2d = bp.reshape(N_PAD // 128, 128)
    g16 = jnp.broadcast_to(gain.astype(F32), (L,))
    b16 = jnp.broadcast_to(bias.astype(F32), (L,))
    part = _stats(xp, b2d)
    outp = _norm(xp, bp, part, g16, b16)
    return outp[:N].reshape(N, 1)


# speedup vs baseline: 125.2519x; 1.0684x over previous
import functools

import jax
import jax.numpy as jnp
from jax import lax
from jax.experimental import pallas as pl
from jax.experimental.pallas import tpu as pltpu
from jax.experimental.pallas import tpu_sc as plsc

N = 1_600_000
NUM_SEG = 50_000
EPS = 0.001

NC = 2
NS = 16
NW = NC * NS
L = 16

SEG_PAD = 51_200
SEG_SLICE = SEG_PAD // NS
N_PAD = 1_638_400
TILE = 2048
ROWS = TILE // 128
K_STEPS = N_PAD // (NW * TILE)
SUB = 800
F32 = jnp.float32
I32 = jnp.int32

_mesh = plsc.VectorSubcoreMesh(core_axis_name="c", subcore_axis_name="s")


def _fill(ref, n, val, dtype):
    v = jnp.full((L,), val, dtype)

    def body(i, _):
        ref[pl.ds(i * L, L)] = v
        return 0

    lax.fori_loop(0, n // L, body, 0)


@functools.partial(
    pl.kernel,
    out_type=jax.ShapeDtypeStruct((NC * 3 * SEG_PAD,), F32),
    mesh=_mesh,
    scratch_types=[
        pltpu.VMEM((TILE,), F32),
        pltpu.VMEM((TILE,), F32),
        pltpu.VMEM((ROWS, 128), I32),
        pltpu.VMEM((128,), F32),
        pltpu.VMEM((SEG_SLICE,), F32),
        pltpu.VMEM_SHARED((SEG_PAD,), F32),
        pltpu.VMEM_SHARED((SEG_PAD,), F32),
        pltpu.VMEM_SHARED((SEG_PAD,), F32),
        pltpu.SemaphoreType.DMA,
    ],
)
def _stats(x_hbm, b2d_hbm, part_hbm, xbuf, xsq, idxbuf, ones, zbuf,
           acc_c, acc_s, acc_q, sem):
    c = lax.axis_index("c")
    s = lax.axis_index("s")
    w = c * NS + s
    base = s * SEG_SLICE

    _fill(ones, 128, 1.0, F32)
    _fill(zbuf, SEG_SLICE, 0.0, F32)
    pltpu.sync_copy(zbuf, acc_c.at[pl.ds(base, SEG_SLICE)])
    pltpu.sync_copy(zbuf, acc_s.at[pl.ds(base, SEG_SLICE)])
    pltpu.sync_copy(zbuf, acc_q.at[pl.ds(base, SEG_SLICE)])
    plsc.subcore_barrier()

    def kstep(k, _):
        e0 = (w * K_STEPS + k) * TILE
        r0 = (w * K_STEPS + k) * ROWS
        pltpu.sync_copy(x_hbm.at[pl.ds(e0, TILE)], xbuf)
        pltpu.sync_copy(b2d_hbm.at[pl.ds(r0, ROWS)], idxbuf)

        def sq(i, _):
            v = xbuf[pl.ds(i * L, L)]
            xsq[pl.ds(i * L, L)] = v * v
            return 0

        lax.fori_loop(0, TILE // L, sq, 0)

        def fire(j, _):
            idxr = idxbuf.at[j]
            pltpu.async_copy(ones, acc_c.at[idxr], sem, add=True)
            pltpu.async_copy(xbuf.at[pl.ds(j * 128, 128)], acc_s.at[idxr], sem, add=True)
            pltpu.async_copy(xsq.at[pl.ds(j * 128, 128)], acc_q.at[idxr], sem, add=True)
            return 0

        lax.fori_loop(0, ROWS, fire, 0)

        def drain(j, _):
            idxr = idxbuf.at[j]
            pltpu.make_async_copy(ones, acc_c.at[idxr], sem).wait()
            pltpu.make_async_copy(xbuf.at[pl.ds(j * 128, 128)], acc_s.at[idxr], sem).wait()
            pltpu.make_async_copy(xsq.at[pl.ds(j * 128, 128)], acc_q.at[idxr], sem).wait()
            return 0

        lax.fori_loop(0, ROWS, drain, 0)
        return 0

    lax.fori_loop(0, K_STEPS, kstep, 0)
    plsc.subcore_barrier()

    pltpu.sync_copy(acc_c.at[pl.ds(base, SEG_SLICE)], zbuf)
    pltpu.sync_copy(zbuf, part_hbm.at[pl.ds(c * 3 * SEG_PAD + base, SEG_SLICE)])
    pltpu.sync_copy(acc_s.at[pl.ds(base, SEG_SLICE)], zbuf)
    pltpu.sync_copy(zbuf, part_hbm.at[pl.ds((c * 3 + 1) * SEG_PAD + base, SEG_SLICE)])
    pltpu.sync_copy(acc_q.at[pl.ds(base, SEG_SLICE)], zbuf)
    pltpu.sync_copy(zbuf, part_hbm.at[pl.ds((c * 3 + 2) * SEG_PAD + base, SEG_SLICE)])


def _rsqrt(v):
    bits = lax.bitcast_convert_type(v, I32)
    magic = jnp.full((L,), 0x5F3759DF, I32)
    shift = jnp.full((L,), 1, I32)
    y = lax.bitcast_convert_type(magic - lax.shift_right_logical(bits, shift), F32)
    half = jnp.float32(0.5)
    three_half = jnp.float32(1.5)
    for _ in range(3):
        y = y * (three_half - half * v * y * y)
    return y


@functools.partial(
    pl.kernel,
    out_type=jax.ShapeDtypeStruct((N_PAD,), F32),
    mesh=_mesh,
    scratch_types=[
        pltpu.VMEM((SEG_SLICE,), I32),
        pltpu.VMEM((SUB,), F32),
        pltpu.VMEM((SUB,), F32),
        pltpu.VMEM((SUB,), F32),
        pltpu.VMEM((SUB,), F32),
        pltpu.VMEM((SUB,), F32),
        pltpu.VMEM((SUB,), F32),
        pltpu.VMEM((TILE,), F32),
        pltpu.VMEM((TILE,), I32),
        pltpu.VMEM((TILE,), I32),
        pltpu.VMEM((TILE,), F32),
        pltpu.VMEM((L,), F32),
        pltpu.VMEM((L,), F32),
        pltpu.VMEM_SHARED((SEG_PAD,), I32),
        pltpu.SemaphoreType.DMA,
    ],
)
def _norm(x_hbm, b_hbm, part_hbm, gain_hbm, bias_hbm, out_hbm,
          ab_v, p0, p1, p2, p3, p4, p5, xbuf, idxbuf, pbuf, obuf,
          gb, bb, ab_sp, sem):
    c = lax.axis_index("c")
    s = lax.axis_index("s")
    w = c * NS + s

    pltpu.sync_copy(gain_hbm, gb)
    pltpu.sync_copy(bias_hbm, bb)
    gain = gb[...]
    bias = bb[...]

    for sub in range(SEG_SLICE // SUB):
        sb = s * SEG_SLICE + sub * SUB
        pltpu.sync_copy(part_hbm.at[pl.ds(0 * SEG_PAD + sb, SUB)], p0)
        pltpu.sync_copy(part_hbm.at[pl.ds(3 * SEG_PAD + sb, SUB)], p1)
        pltpu.sync_copy(part_hbm.at[pl.ds(1 * SEG_PAD + sb, SUB)], p2)
        pltpu.sync_copy(part_hbm.at[pl.ds(4 * SEG_PAD + sb, SUB)], p3)
        pltpu.sync_copy(part_hbm.at[pl.ds(2 * SEG_PAD + sb, SUB)], p4)
        pltpu.sync_copy(part_hbm.at[pl.ds(5 * SEG_PAD + sb, SUB)], p5)

        def seg(i, _):
            d = pl.ds(i * L, L)
            cnt = p0[d] + p1[d]
            cl = jnp.maximum(cnt, jnp.float32(1.0))
            sm = p2[d] + p3[d]
            q = p4[d] + p5[d]
            m = sm / cl
            var = jnp.maximum(q / cl - m * m, jnp.float32(0.0))
            vs = jnp.maximum(var, jnp.float32(1e-30))
            std = vs * _rsqrt(vs)
            a = gain / (std + jnp.float32(EPS))
            b = bias - m * a
            half = jnp.full((L,), 0x8000, I32)
            himask = jnp.full((L,), -65536, I32)
            s16 = jnp.full((L,), 16, I32)
            ar = lax.bitcast_convert_type(a, I32) + half
            br = lax.bitcast_convert_type(b, I32) + half
            packed = (ar & himask) | lax.shift_right_logical(br, s16)
            ab_v[pl.ds(sub * SUB + i * L, L)] = packed
            return 0

        lax.fori_loop(0, SUB // L, seg, 0)

    sl = pl.ds(s * SEG_SLICE, SEG_SLICE)
    pltpu.sync_copy(ab_v, ab_sp.at[sl])
    plsc.subcore_barrier()

    def kstep(k, _):
        e0 = (w * K_STEPS + k) * TILE
        pltpu.sync_copy(x_hbm.at[pl.ds(e0, TILE)], xbuf)
        pltpu.sync_copy(b_hbm.at[pl.ds(e0, TILE)], idxbuf)

        def fire(j, _):
            d = pl.ds(j * 128, 128)
            idxr = idxbuf.at[d]
            pltpu.async_copy(ab_sp.at[idxr], pbuf.at[d], sem)
            return 0

        lax.fori_loop(0, ROWS, fire, 0)

        def drain(j, _):
            d = pl.ds(j * 128, 128)
            idxr = idxbuf.at[d]
            pltpu.make_async_copy(ab_sp.at[idxr], pbuf.at[d], sem).wait()
            return 0

        lax.fori_loop(0, ROWS, drain, 0)

        himask = jnp.full((L,), -65536, I32)
        s16 = jnp.full((L,), 16, I32)

        def inner(i, _):
            d = pl.ds(i * L, L)
            p = pbuf[d]
            a = lax.bitcast_convert_type(p & himask, F32)
            b = lax.bitcast_convert_type(lax.shift_left(p, s16), F32)
            obuf[d] = xbuf[d] * a + b
            return 0

        lax.fori_loop(0, TILE // L, inner, 0)
        pltpu.sync_copy(obuf, out_hbm.at[pl.ds(e0, TILE)])
        return 0

    lax.fori_loop(0, K_STEPS, kstep, 0)


def kernel(inputs, batch, gain, bias):
    x = inputs.astype(F32)
    b = batch.astype(I32)
    pad = N_PAD - N
    xp = jnp.concatenate([x, jnp.zeros((pad,), F32)])
    dummy = NUM_SEG + (jnp.arange(pad, dtype=I32) % (SEG_PAD - NUM_SEG))
    bp = jnp.concatenate([b, dummy])
    b2d = bp.reshape(N_PAD // 128, 128)
    g16 = jnp.broadcast_to(gain.astype(F32), (L,))
    b16 = jnp.broadcast_to(bias.astype(F32), (L,))
    part = _stats(xp, b2d)
    outp = _norm(xp, bp, part, g16, b16)
    return outp[:N].reshape(N, 1)
